# baseline (speedup 1.0000x reference)
import functools

import jax
import jax.numpy as jnp
from jax import lax
from jax.experimental import pallas as pl
from jax.experimental.pallas import tpu as pltpu
from jax.experimental.pallas import tpu_sc as plsc

_NUM_CORES = 2
_NUM_SUBCORES = 16
_NW = _NUM_CORES * _NUM_SUBCORES

_BB = 128


def _build(B, D, L):
    DT = D // 8
    NBT = B // _BB
    TW = DT * 8 * _BB

    mesh = plsc.VectorSubcoreMesh(
        core_axis_name="c", subcore_axis_name="s")

    @functools.partial(
        pl.kernel,
        out_type=jax.ShapeDtypeStruct((L, DT, NBT, 8 * _BB), jnp.float32),
        mesh=mesh,
        scratch_types=[
            pltpu.VMEM((L, _BB), jnp.int32),
            pltpu.VMEM((L, D), jnp.float32),
            [pltpu.VMEM((_BB, D), jnp.float32) for _ in range(2)],
            [pltpu.VMEM((TW,), jnp.float32) for _ in range(2)],
            [pltpu.SemaphoreType.DMA for _ in range(2)],
            [pltpu.SemaphoreType.DMA for _ in range(2)],
        ],
        compiler_params=pltpu.CompilerParams(
            use_tc_tiling_on_sc=False, needs_layout_passes=False),
    )
    def emb(idx_hbm, tok_hbm, pos_hbm, out_hbm,
            idx_v, pos_v, gs, ts, gsems, osems):
        wid = lax.axis_index("s") * _NUM_CORES + lax.axis_index("c")
        pltpu.sync_copy(idx_hbm.at[wid], idx_v)
        pltpu.sync_copy(pos_hbm, pos_v)

        lanes = lax.iota(jnp.int32, 16)
        cflat = [((lanes + 16 * j) >> 3) * (8 * _BB)
                 + ((lanes + 16 * j) & 7) * _BB
                 for j in range(D // 16)]

        def fire_gather(l, p):
            pltpu.async_copy(tok_hbm.at[idx_v.at[l]], gs[p], gsems[p])

        def wait_gather(p):
            pltpu.make_async_copy(
                tok_hbm.at[pl.ds(0, _BB)], gs[p], gsems[p]).wait()

        def fire_out(l, p):
            for dt in range(DT):
                pltpu.async_copy(
                    ts[p].at[pl.ds(dt * 8 * _BB, 8 * _BB)],
                    out_hbm.at[l, dt, wid], osems[p])

        def wait_out(p, l_fired):
            for dt in range(DT):
                pltpu.make_async_copy(
                    ts[p].at[pl.ds(dt * 8 * _BB, 8 * _BB)],
                    out_hbm.at[l_fired, dt, wid], osems[p]).wait()

        fire_gather(0, 0)

        def pair(h, carry):
            for p in range(2):
                l = h * 2 + p

                @pl.when(l + 1 < L)
                def _():
                    fire_gather(l + 1, 1 - p)

                wait_gather(p)

                @pl.when(l >= 2)
                def _():
                    wait_out(p, l - 2)

                pos_j = [pos_v[l, pl.ds(16 * j, 16)] for j in range(D // 16)]

                @plsc.parallel_loop(0, _BB, unroll=4)
                def _(b):
                    cb = lanes * 0 + b
                    for j in range(D // 16):
                        x = gs[p][b, pl.ds(16 * j, 16)] + pos_j[j]
                        plsc.store_scatter(ts[p], [cflat[j] + cb], x)

                fire_out(l, p)
            return carry

        lax.fori_loop(0, L // 2, pair, 0)
        wait_out(0, L - 2)
        wait_out(1, L - 1)

    return emb


def kernel(inputs, token_table, pos_table):
    B, L = inputs.shape
    _, D = token_table.shape
    idxr = inputs.reshape(B // _BB, _BB, L).transpose(0, 2, 1)
    emb = _build(B, D, L)
    out4 = emb(idxr, token_table, pos_table)
    out5 = out4.reshape(L, D // 8, B // _BB, 8, _BB)
    return out5.transpose(2, 4, 0, 1, 3).reshape(B, L, D)


# speedup vs baseline: 2.8833x; 2.8833x over previous
import functools

import jax
import jax.numpy as jnp
from jax import lax
from jax.experimental import pallas as pl
from jax.experimental.pallas import tpu as pltpu
from jax.experimental.pallas import tpu_sc as plsc

_NUM_CORES = 2
_NUM_SUBCORES = 16
_NW = _NUM_CORES * _NUM_SUBCORES

_BB = 128
_PITCH = 65


def _build(B, D, L):
    DT = D // 8
    NBT = B // _BB

    mesh = plsc.VectorSubcoreMesh(
        core_axis_name="c", subcore_axis_name="s")

    @functools.partial(
        pl.kernel,
        out_type=jax.ShapeDtypeStruct((L, DT, NBT, 8, _BB), jnp.float32),
        mesh=mesh,
        scratch_types=[
            pltpu.VMEM((L, _BB), jnp.int32),
            pltpu.VMEM((L, D), jnp.float32),
            [pltpu.VMEM((_BB, D), jnp.float32) for _ in range(2)],
            pltpu.VMEM((_BB * _PITCH,), jnp.float32),
            [pltpu.VMEM((D, _BB), jnp.float32) for _ in range(2)],
            [pltpu.SemaphoreType.DMA for _ in range(2)],
            [pltpu.SemaphoreType.DMA for _ in range(2)],
        ],
        compiler_params=pltpu.CompilerParams(
            use_tc_tiling_on_sc=False, needs_layout_passes=False),
    )
    def emb(idx_hbm, tok_hbm, pos_hbm, out_hbm,
            idx_v, pos_v, gs, skew, ts, gsems, osems):
        wid = lax.axis_index("s") * _NUM_CORES + lax.axis_index("c")
        pltpu.sync_copy(idx_hbm.at[wid], idx_v)
        pltpu.sync_copy(pos_hbm, pos_v)

        lanes = lax.iota(jnp.int32, 16)
        csk = [lanes + 16 * j for j in range(D // 16)]
        cg2 = [(lanes + 16 * j) * _PITCH for j in range(_BB // 16)]

        def fire_gather(l, p):
            pltpu.async_copy(tok_hbm.at[idx_v.at[l]], gs[p], gsems[p])

        def wait_gather(p):
            pltpu.make_async_copy(
                tok_hbm.at[pl.ds(0, _BB)], gs[p], gsems[p]).wait()

        def fire_out(l, p):
            for dt in range(DT):
                pltpu.async_copy(
                    ts[p].at[pl.ds(dt * 8, 8)],
                    out_hbm.at[l, dt, wid], osems[p])

        def wait_out(p, l_fired):
            for dt in range(DT):
                pltpu.make_async_copy(
                    ts[p].at[pl.ds(dt * 8, 8)],
                    out_hbm.at[l_fired, dt, wid], osems[p]).wait()

        fire_gather(0, 0)

        def pair(h, carry):
            for p in range(2):
                l = h * 2 + p

                @pl.when(l + 1 < L)
                def _():
                    fire_gather(l + 1, 1 - p)

                wait_gather(p)

                @pl.when(l >= 2)
                def _():
                    wait_out(p, l - 2)

                pos_j = [pos_v[l, pl.ds(16 * j, 16)] for j in range(D // 16)]

                @plsc.parallel_loop(0, _BB, unroll=4)
                def _(b):
                    cb = lanes * 0 + b * _PITCH
                    for j in range(D // 16):
                        x = gs[p][b, pl.ds(16 * j, 16)] + pos_j[j]
                        plsc.store_scatter(skew, [csk[j] + cb], x)

                @plsc.parallel_loop(0, D, unroll=2)
                def _(d):
                    sd = lanes * 0 + d
                    for j2 in range(_BB // 16):
                        xv = plsc.load_gather(skew, [cg2[j2] + sd])
                        ts[p][d, pl.ds(16 * j2, 16)] = xv

                fire_out(l, p)
            return carry

        lax.fori_loop(0, L // 2, pair, 0)
        wait_out(0, L - 2)
        wait_out(1, L - 1)

    return emb


def kernel(inputs, token_table, pos_table):
    B, L = inputs.shape
    _, D = token_table.shape
    idxr = inputs.reshape(B // _BB, _BB, L).transpose(0, 2, 1)
    emb = _build(B, D, L)
    out5 = emb(idxr, token_table, pos_table)
    return out5.transpose(2, 4, 0, 1, 3).reshape(B, L, D)


# speedup vs baseline: 2.9755x; 1.0320x over previous
import functools

import jax
import jax.numpy as jnp
from jax import lax
from jax.experimental import pallas as pl
from jax.experimental.pallas import tpu as pltpu
from jax.experimental.pallas import tpu_sc as plsc

_NUM_CORES = 2
_NUM_SUBCORES = 16
_NW = _NUM_CORES * _NUM_SUBCORES

_BB = 128
_PITCH = 129


def _build(B, D, L):
    DT = D // 8
    NBT = B // _BB

    mesh = plsc.VectorSubcoreMesh(
        core_axis_name="c", subcore_axis_name="s")

    @functools.partial(
        pl.kernel,
        out_type=jax.ShapeDtypeStruct((L, DT, NBT, 8, _BB), jnp.float32),
        mesh=mesh,
        scratch_types=[
            pltpu.VMEM((L, _BB), jnp.int32),
            pltpu.VMEM((L, D), jnp.float32),
            [pltpu.VMEM((_BB, D), jnp.float32) for _ in range(2)],
            [pltpu.VMEM((D, _PITCH), jnp.float32) for _ in range(2)],
            [pltpu.SemaphoreType.DMA for _ in range(2)],
            [pltpu.SemaphoreType.DMA for _ in range(2)],
        ],
        compiler_params=pltpu.CompilerParams(
            use_tc_tiling_on_sc=False, needs_layout_passes=False),
    )
    def emb(idx_hbm, tok_hbm, pos_hbm, out_hbm,
            idx_v, pos_v, gs, ts, gsems, osems):
        wid = lax.axis_index("s") * _NUM_CORES + lax.axis_index("c")
        pltpu.sync_copy(idx_hbm.at[wid], idx_v)
        pltpu.sync_copy(pos_hbm, pos_v)

        lanes = lax.iota(jnp.int32, 16)
        crow = [lanes + 16 * j for j in range(D // 16)]

        def fire_gather(l, p):
            pltpu.async_copy(tok_hbm.at[idx_v.at[l]], gs[p], gsems[p])

        def wait_gather(p):
            pltpu.make_async_copy(
                tok_hbm.at[pl.ds(0, _BB)], gs[p], gsems[p]).wait()

        def fire_out(l, p):
            for dt in range(DT):
                pltpu.async_copy(
                    ts[p].at[pl.ds(dt * 8, 8), pl.ds(0, _BB)],
                    out_hbm.at[l, dt, wid], osems[p])

        def wait_out(p, l_fired):
            for dt in range(DT):
                pltpu.make_async_copy(
                    ts[p].at[pl.ds(dt * 8, 8), pl.ds(0, _BB)],
                    out_hbm.at[l_fired, dt, wid], osems[p]).wait()

        fire_gather(0, 0)

        def pair(h, carry):
            for p in range(2):
                l = h * 2 + p

                @pl.when(l + 1 < L)
                def _():
                    fire_gather(l + 1, 1 - p)

                wait_gather(p)

                @pl.when(l >= 2)
                def _():
                    wait_out(p, l - 2)

                pos_j = [pos_v[l, pl.ds(16 * j, 16)] for j in range(D // 16)]

                @plsc.parallel_loop(0, _BB, unroll=4)
                def _(b):
                    cb = lanes * 0 + b
                    for j in range(D // 16):
                        x = gs[p][b, pl.ds(16 * j, 16)] + pos_j[j]
                        plsc.store_scatter(ts[p], [crow[j], cb], x)

                fire_out(l, p)
            return carry

        lax.fori_loop(0, L // 2, pair, 0)
        wait_out(0, L - 2)
        wait_out(1, L - 1)

    return emb


def kernel(inputs, token_table, pos_table):
    B, L = inputs.shape
    _, D = token_table.shape
    idxr = inputs.reshape(B // _BB, _BB, L).transpose(0, 2, 1)
    emb = _build(B, D, L)
    out5 = emb(idxr, token_table, pos_table)
    return out5.transpose(2, 4, 0, 1, 3).reshape(B, L, D)
